# baseline (speedup 1.0000x reference)
import functools

import jax
import jax.numpy as jnp
from jax import lax
from jax.experimental import pallas as pl
from jax.experimental.pallas import tpu as pltpu
from jax.experimental.pallas import tpu_sc as plsc


def _sc_broadcast_copy(B, M, D, dtype):
    NC, NS = 2, 16
    NW = NC * NS
    rows_per_w = M // NW
    C = 32
    n_chunks = rows_per_w // C

    mesh = plsc.VectorSubcoreMesh(core_axis_name="c", subcore_axis_name="s")

    @functools.partial(
        pl.kernel,
        out_type=jax.ShapeDtypeStruct((B, M, D), dtype),
        mesh=mesh,
        scratch_types=[
            pltpu.VMEM((2, C, D), dtype),
            pltpu.SemaphoreType.DMA((2,)),
            pltpu.SemaphoreType.DMA((2,)),
        ],
    )
    def copy_kernel(table_hbm, out_hbm, buf, in_sem, out_sem):
        wid = lax.axis_index("s") * NC + lax.axis_index("c")
        base = wid * rows_per_w

        def load(i, slot):
            return pltpu.make_async_copy(
                table_hbm.at[pl.ds(base + i * C, C)],
                buf.at[slot],
                in_sem.at[slot],
            )

        def store(i, slot, b):
            return pltpu.make_async_copy(
                buf.at[slot],
                out_hbm.at[b, pl.ds(base + i * C, C)],
                out_sem.at[slot],
            )

        load(0, 0).start()
        for i in range(n_chunks):
            s = i % 2
            if i + 1 < n_chunks:
                if i >= 1:
                    for b in range(B):
                        store(i - 1, 1 - s, b).wait()
                load(i + 1, 1 - s).start()
            load(i, s).wait()
            for b in range(B):
                store(i, s, b).start()
        for i in range(max(n_chunks - 2, 0), n_chunks):
            for b in range(B):
                store(i, i % 2, b).wait()

    return copy_kernel


def kernel(x, table):
    B = x.shape[0]
    M, D = table.shape
    return _sc_broadcast_copy(B, M, D, table.dtype)(table)
